# baseline (speedup 1.0000x reference)
import jax
import jax.numpy as jnp
from jax.experimental import pallas as pl
from jax.experimental.pallas import tpu as pltpu

_B = 8
_TOTAL = 16384
_D = 4096
_MAXCHUNK = 512
_NBUF = 7

_SIZES = [128, 256] + [512] * 31 + [128]
_OFFS = [0]
for _s in _SIZES[:-1]:
    _OFFS.append(_OFFS[-1] + _s)
assert _OFFS[-1] + _SIZES[-1] == _TOTAL
_NCHUNK = len(_SIZES)


def _copy_cu_kernel(len_ref, in_ref, out_ref, cu_ref, buf, rsem, wsem):
    cu_ref[0] = jnp.int32(0)
    acc = jnp.int32(0)
    for i in range(_B):
        acc = acc + len_ref[i]
        cu_ref[i + 1] = acc

    def rd(c):
        b = c % _NBUF
        return pltpu.make_async_copy(
            in_ref.at[pl.ds(_OFFS[c], _SIZES[c]), :],
            buf.at[b, pl.ds(0, _SIZES[c]), :],
            rsem.at[b],
        )

    def wr(c):
        b = c % _NBUF
        return pltpu.make_async_copy(
            buf.at[b, pl.ds(0, _SIZES[c]), :],
            out_ref.at[pl.ds(_OFFS[c], _SIZES[c]), :],
            wsem.at[b],
        )

    for c in range(_NBUF):
        rd(c).start()
    for c in range(_NCHUNK):
        rd(c).wait()
        wr(c).start()
        nc = c + _NBUF
        if nc < _NCHUNK:
            wr(c).wait()
            rd(nc).start()
    for c in range(_NCHUNK - _NBUF, _NCHUNK):
        wr(c).wait()


def kernel(hidden_states, lengths_cpu):
    lengths = lengths_cpu.astype(jnp.int32)
    values, cu_lengths = pl.pallas_call(
        _copy_cu_kernel,
        in_specs=[
            pl.BlockSpec(memory_space=pltpu.SMEM),
            pl.BlockSpec(memory_space=pl.ANY),
        ],
        out_specs=[
            pl.BlockSpec(memory_space=pl.ANY),
            pl.BlockSpec(memory_space=pltpu.SMEM),
        ],
        out_shape=[
            jax.ShapeDtypeStruct((_TOTAL, _D), jnp.float32),
            jax.ShapeDtypeStruct((_B + 1,), jnp.int32),
        ],
        scratch_shapes=[
            pltpu.VMEM((_NBUF, _MAXCHUNK, _D), jnp.float32),
            pltpu.SemaphoreType.DMA((_NBUF,)),
            pltpu.SemaphoreType.DMA((_NBUF,)),
        ],
    )(lengths, hidden_states)
    return values, cu_lengths


# speedup vs baseline: 1.0060x; 1.0060x over previous
import jax
import jax.numpy as jnp
from jax.experimental import pallas as pl
from jax.experimental.pallas import tpu as pltpu

_B = 8
_TOTAL = 16384
_D = 4096
_MAXCHUNK = 512
_NBUF = 7

_SIZES = [_MAXCHUNK] * (_TOTAL // _MAXCHUNK)
_OFFS = [0]
for _s in _SIZES[:-1]:
    _OFFS.append(_OFFS[-1] + _s)
assert _OFFS[-1] + _SIZES[-1] == _TOTAL
_NCHUNK = len(_SIZES)


def _copy_cu_kernel(len_ref, in_ref, out_ref, cu_ref, buf, rsem, wsem):
    cu_ref[0] = jnp.int32(0)
    acc = jnp.int32(0)
    for i in range(_B):
        acc = acc + len_ref[i]
        cu_ref[i + 1] = acc

    def rd(c):
        b = c % _NBUF
        return pltpu.make_async_copy(
            in_ref.at[pl.ds(_OFFS[c], _SIZES[c]), :],
            buf.at[b, pl.ds(0, _SIZES[c]), :],
            rsem.at[b],
        )

    def wr(c):
        b = c % _NBUF
        return pltpu.make_async_copy(
            buf.at[b, pl.ds(0, _SIZES[c]), :],
            out_ref.at[pl.ds(_OFFS[c], _SIZES[c]), :],
            wsem.at[b],
        )

    for c in range(_NBUF):
        rd(c).start()
    for c in range(_NCHUNK):
        rd(c).wait()
        wr(c).start()
        nc = c + _NBUF
        if nc < _NCHUNK:
            wr(c).wait()
            rd(nc).start()
    for c in range(_NCHUNK - _NBUF, _NCHUNK):
        wr(c).wait()


def kernel(hidden_states, lengths_cpu):
    lengths = lengths_cpu.astype(jnp.int32)
    values, cu_lengths = pl.pallas_call(
        _copy_cu_kernel,
        in_specs=[
            pl.BlockSpec(memory_space=pltpu.SMEM),
            pl.BlockSpec(memory_space=pl.ANY),
        ],
        out_specs=[
            pl.BlockSpec(memory_space=pl.ANY),
            pl.BlockSpec(memory_space=pltpu.SMEM),
        ],
        out_shape=[
            jax.ShapeDtypeStruct((_TOTAL, _D), jnp.float32),
            jax.ShapeDtypeStruct((_B + 1,), jnp.int32),
        ],
        scratch_shapes=[
            pltpu.VMEM((_NBUF, _MAXCHUNK, _D), jnp.float32),
            pltpu.SemaphoreType.DMA((_NBUF,)),
            pltpu.SemaphoreType.DMA((_NBUF,)),
        ],
    )(lengths, hidden_states)
    return values, cu_lengths
